# Initial kernel scaffold; baseline (speedup 1.0000x reference)
import jax
import jax.numpy as jnp
from jax.experimental import pallas as pl


def kernel(x_symptom, x_disease, edge_src_sd, edge_dst_sd, edge_src_ds, edge_dst_ds, W_src_sd, W_dst_sd, att_src_sd, att_dst_sd, bias_sd, W_src_ds, W_dst_ds, att_src_ds, att_dst_ds, bias_ds, W_lin, b_lin):
    raise NotImplementedError("write your pallas kernel here")


# speedup vs baseline: 1.5308x; 1.5308x over previous
import jax
import jax.numpy as jnp
from jax.experimental import pallas as pl

NS = 50000
ND = 10000
HID = 128
OUT = 128

_R = 400


def _epilogue_body(num_ref, den_ref, bias_ref, wlin_ref, blin_ref, out_ref):
    h = num_ref[...] / (den_ref[...] + 1e-16) + bias_ref[...]
    h = jnp.maximum(h, 0.0)
    out_ref[...] = jnp.dot(h, wlin_ref[...],
                           preferred_element_type=jnp.float32) + blin_ref[...]


def _epilogue(num, denom, bias_ds, W_lin, b_lin):
    grid = (NS // _R,)
    return pl.pallas_call(
        _epilogue_body,
        grid=grid,
        in_specs=[
            pl.BlockSpec((_R, HID), lambda i: (i, 0)),
            pl.BlockSpec((_R, 1), lambda i: (i, 0)),
            pl.BlockSpec((1, HID), lambda i: (0, 0)),
            pl.BlockSpec((HID, OUT), lambda i: (0, 0)),
            pl.BlockSpec((1, OUT), lambda i: (0, 0)),
        ],
        out_specs=pl.BlockSpec((_R, OUT), lambda i: (i, 0)),
        out_shape=jax.ShapeDtypeStruct((NS, OUT), jnp.float32),
    )(num, denom.reshape(NS, 1), bias_ds.reshape(1, HID), W_lin,
      b_lin.reshape(1, OUT))


def kernel(x_symptom, x_disease, edge_src_sd, edge_dst_sd, edge_src_ds,
           edge_dst_ds, W_src_sd, W_dst_sd, att_src_sd, att_dst_sd, bias_sd,
           W_src_ds, W_dst_ds, att_src_ds, att_dst_ds, bias_ds, W_lin, b_lin):
    h_src = x_disease @ W_src_ds
    a_src = h_src @ att_src_ds
    a_dst = x_symptom @ (W_dst_ds @ att_dst_ds)

    alpha = a_src[edge_src_ds] + a_dst[edge_dst_ds]
    alpha = jax.nn.leaky_relu(alpha, negative_slope=0.2)
    ex = jnp.exp(alpha)
    denom = jax.ops.segment_sum(ex, edge_dst_ds, num_segments=NS)
    num = jax.ops.segment_sum(h_src[edge_src_ds] * ex[:, None], edge_dst_ds,
                              num_segments=NS)
    return _epilogue(num, denom, bias_ds, W_lin, b_lin)


# speedup vs baseline: 10.9104x; 7.1275x over previous
import functools

import jax
import jax.numpy as jnp
from jax import lax
from jax.experimental import pallas as pl
from jax.experimental.pallas import tpu as pltpu
from jax.experimental.pallas import tpu_sc as plsc

NS = 50000
ND = 10000
HID = 128
OUT = 128
E = 320000

NC = 2
NT = 16
CW = 32
NCH = HID // CW
EPAD = NC * 0 + NT * 160 * 128
ROWS = EPAD // 128
TROWS = ROWS // NT
BLK = 8
NBLK = TROWS // BLK
DSTRIPE = 2000
NDSTRIPE = NS // DSTRIPE
NBOUNCE = 200
NAP = 12000
SUB = 2

_R = 400


def _pre_body(x_ref, w_ref, att_ref, h_ref, a_ref):
    h = jnp.dot(x_ref[...], w_ref[...], preferred_element_type=jnp.float32)
    h_ref[...] = h
    a_ref[...] = jnp.dot(h, att_ref[...], preferred_element_type=jnp.float32)


def _pre(x_disease, W_src_ds, att_src_ds):
    return pl.pallas_call(
        _pre_body,
        grid=(ND // _R,),
        in_specs=[
            pl.BlockSpec((_R, HID), lambda i: (i, 0)),
            pl.BlockSpec((HID, HID), lambda i: (0, 0)),
            pl.BlockSpec((HID, 1), lambda i: (0, 0)),
        ],
        out_specs=[
            pl.BlockSpec((_R, HID), lambda i: (i, 0)),
            pl.BlockSpec((_R, 1), lambda i: (i, 0)),
        ],
        out_shape=[
            jax.ShapeDtypeStruct((ND, HID), jnp.float32),
            jax.ShapeDtypeStruct((ND, 1), jnp.float32),
        ],
    )(x_disease, W_src_ds, att_src_ds.reshape(HID, 1))


def _adst_body(x_ref, w_ref, att_ref, a_ref):
    v = jnp.dot(w_ref[...], att_ref[...], preferred_element_type=jnp.float32)
    a_ref[...] = jnp.dot(x_ref[...], v, preferred_element_type=jnp.float32)


def _adst(x_symptom, W_dst_ds, att_dst_ds):
    return pl.pallas_call(
        _adst_body,
        grid=(NS // _R,),
        in_specs=[
            pl.BlockSpec((_R, HID), lambda i: (i, 0)),
            pl.BlockSpec((HID, HID), lambda i: (0, 0)),
            pl.BlockSpec((HID, 1), lambda i: (0, 0)),
        ],
        out_specs=pl.BlockSpec((_R, 1), lambda i: (i, 0)),
        out_shape=jax.ShapeDtypeStruct((NS, 1), jnp.float32),
    )(x_symptom, W_dst_ds, att_dst_ds.reshape(HID, 1))


def _sc_body(src_hbm, dst_hbm, hflat_hbm, asrc_hbm, adst_hbm, zn_hbm, zd_hbm,
             num_hbm, den_hbm,
             srcb, dstb, exb, avb, dvb, rows_v, nbounce, dbounce,
             asrc_sh, adst_sh, num_sh, den_sh, sem):
    c = lax.axis_index("c")
    s = lax.axis_index("s")

    for st in range(NAP // DSTRIPE):
        @pl.when(s == st)
        def _():
            pltpu.sync_copy(asrc_hbm.at[pl.ds(st * DSTRIPE, DSTRIPE)],
                            dbounce)
            pltpu.sync_copy(dbounce,
                            asrc_sh.at[pl.ds(st * DSTRIPE, DSTRIPE)])
    for st in range(NDSTRIPE):
        @pl.when(s == ((st + NAP // DSTRIPE) % NT))
        def _():
            pltpu.sync_copy(adst_hbm.at[pl.ds(st * DSTRIPE, DSTRIPE)],
                            dbounce)
            pltpu.sync_copy(dbounce,
                            adst_sh.at[pl.ds(st * DSTRIPE, DSTRIPE)])

    def kloop(k, kcarry):
        g = c * 2 + k
        goff = g * ND

        pltpu.sync_copy(zn_hbm, nbounce)
        @pl.when(k == 0)
        def _():
            pltpu.sync_copy(zd_hbm, dbounce)
        for st in range(NDSTRIPE):
            @pl.when(s == (st % NT))
            def _():
                for i in range(DSTRIPE // NBOUNCE):
                    pltpu.sync_copy(
                        nbounce,
                        num_sh.at[pl.ds(st * DSTRIPE + i * NBOUNCE, NBOUNCE)])
                @pl.when((k == 0) & (c == 0))
                def _():
                    pltpu.sync_copy(
                        dbounce, den_sh.at[pl.ds(st * DSTRIPE, DSTRIPE)])
        plsc.subcore_barrier()

        row0 = s * TROWS

        def block(b, carry):
            r0 = row0 + b * BLK
            pltpu.sync_copy(src_hbm.at[pl.ds(r0, BLK)], srcb)
            pltpu.sync_copy(dst_hbm.at[pl.ds(r0, BLK)], dstb)

            descs = []
            for r in range(BLK):
                descs.append(pltpu.async_copy(
                    asrc_sh.at[srcb.at[r]], avb.at[r], sem))
                descs.append(pltpu.async_copy(
                    adst_sh.at[dstb.at[r]], dvb.at[r], sem))
            for d in descs:
                d.wait()

            def grp(q, cr2):
                r = q // 8
                t16 = (q % 8) * 16
                al = avb[r, pl.ds(t16, 16)] + dvb[r, pl.ds(t16, 16)]
                al = jnp.where(al >= 0, al, al * jnp.float32(0.2))
                exb[r, pl.ds(t16, 16)] = jnp.exp(al)
                sv = srcb[r, pl.ds(t16, 16)]
                srcb[r, pl.ds(t16, 16)] = (
                    jnp.minimum(sv, jnp.int32(ND - 1)) + goff)
                return cr2
            lax.fori_loop(0, BLK * 8, grp, 0)

            def sbody(sb, cr):
                ds2 = [
                    pltpu.async_copy(hflat_hbm.at[srcb.at[sb * SUB + i]],
                                     rows_v.at[i], sem)
                    for i in range(SUB)
                ]
                for d in ds2:
                    d.wait()
                for i in range(SUB):
                    def sgrp(t, cr2, i=i):
                        r = sb * SUB + i
                        t16 = t * 16
                        e16 = exb[r, pl.ds(t16, 16)]
                        for j in range(16):
                            ej = e16.at[jnp.full((16,), j, jnp.int32)].get(
                                mode="promise_in_bounds")
                            for h in range(CW // 16):
                                x = rows_v[i, t16 + j, pl.ds(h * 16, 16)]
                                rows_v[i, t16 + j, pl.ds(h * 16, 16)] = x * ej
                        return cr2
                    lax.fori_loop(0, 8, sgrp, 0)
                for i in range(SUB):
                    pltpu.sync_copy(rows_v.at[i],
                                    num_sh.at[dstb.at[sb * SUB + i]],
                                    add=True)
                return cr
            lax.fori_loop(0, BLK // SUB, sbody, 0)

            @pl.when((k == 0) & (c == 0))
            def _():
                for r in range(BLK):
                    pltpu.sync_copy(exb.at[r], den_sh.at[dstb.at[r]],
                                    add=True)
            return carry

        lax.fori_loop(0, NBLK, block, 0)
        plsc.subcore_barrier()

        for st in range(NDSTRIPE):
            @pl.when(s == (st % NT))
            def _():
                for i in range(DSTRIPE // NBOUNCE):
                    pltpu.sync_copy(
                        num_sh.at[pl.ds(st * DSTRIPE + i * NBOUNCE, NBOUNCE)],
                        nbounce)
                    pltpu.sync_copy(
                        nbounce,
                        num_hbm.at[pl.ds(
                            g * NS + st * DSTRIPE + i * NBOUNCE, NBOUNCE)])
                @pl.when((k == 0) & (c == 0))
                def _():
                    pltpu.sync_copy(
                        den_sh.at[pl.ds(st * DSTRIPE, DSTRIPE)], dbounce)
                    pltpu.sync_copy(
                        dbounce, den_hbm.at[pl.ds(st * DSTRIPE, DSTRIPE)])
        plsc.subcore_barrier()
        return kcarry

    lax.fori_loop(0, 2, kloop, 0)


_sc_edges = functools.partial(
    pl.kernel,
    out_type=[
        jax.ShapeDtypeStruct((NCH * NS, CW), jnp.float32),
        jax.ShapeDtypeStruct((NS,), jnp.float32),
    ],
    mesh=plsc.VectorSubcoreMesh(core_axis_name="c", subcore_axis_name="s"),
    scratch_types=[
        pltpu.VMEM((BLK, 128), jnp.int32),
        pltpu.VMEM((BLK, 128), jnp.int32),
        pltpu.VMEM((BLK, 128), jnp.float32),
        pltpu.VMEM((BLK, 128), jnp.float32),
        pltpu.VMEM((BLK, 128), jnp.float32),
        pltpu.VMEM((SUB, 128, CW), jnp.float32),
        pltpu.VMEM((NBOUNCE, CW), jnp.float32),
        pltpu.VMEM((DSTRIPE,), jnp.float32),
        pltpu.VMEM_SHARED((NAP,), jnp.float32),
        pltpu.VMEM_SHARED((NS,), jnp.float32),
        pltpu.VMEM_SHARED((NS, CW), jnp.float32),
        pltpu.VMEM_SHARED((NS,), jnp.float32),
        pltpu.SemaphoreType.DMA,
    ],
    compiler_params=pltpu.CompilerParams(needs_layout_passes=False,
                                         use_tc_tiling_on_sc=False),
)(_sc_body)


def _epilogue_body(n0_ref, n1_ref, n2_ref, n3_ref, den_ref, bias_ref,
                   wlin_ref, blin_ref, out_ref):
    num = jnp.concatenate(
        [n0_ref[...], n1_ref[...], n2_ref[...], n3_ref[...]], axis=1)
    h = num / (den_ref[...] + 1e-16) + bias_ref[...]
    h = jnp.maximum(h, 0.0)
    out_ref[...] = jnp.dot(h, wlin_ref[...],
                           preferred_element_type=jnp.float32) + blin_ref[...]


def _epilogue(num_flat, denom, bias_ds, W_lin, b_lin):
    nspec = [
        pl.BlockSpec((_R, CW), lambda i, j=j: (j * (NS // _R) + i, 0))
        for j in range(NCH)
    ]
    return pl.pallas_call(
        _epilogue_body,
        grid=(NS // _R,),
        in_specs=nspec + [
            pl.BlockSpec((_R, 1), lambda i: (i, 0)),
            pl.BlockSpec((1, HID), lambda i: (0, 0)),
            pl.BlockSpec((HID, OUT), lambda i: (0, 0)),
            pl.BlockSpec((1, OUT), lambda i: (0, 0)),
        ],
        out_specs=pl.BlockSpec((_R, OUT), lambda i: (i, 0)),
        out_shape=jax.ShapeDtypeStruct((NS, OUT), jnp.float32),
    )(num_flat, num_flat, num_flat, num_flat, denom.reshape(NS, 1),
      bias_ds.reshape(1, HID), W_lin, b_lin.reshape(1, OUT))


def kernel(x_symptom, x_disease, edge_src_sd, edge_dst_sd, edge_src_ds,
           edge_dst_ds, W_src_sd, W_dst_sd, att_src_sd, att_dst_sd, bias_sd,
           W_src_ds, W_dst_ds, att_src_ds, att_dst_ds, bias_ds, W_lin, b_lin):
    h_src, a_src = _pre(x_disease, W_src_ds, att_src_ds)
    a_dst = _adst(x_symptom, W_dst_ds, att_dst_ds)

    a_srcp = jnp.pad(a_src.reshape(ND), (0, NAP - ND), constant_values=-1e30)
    h_flat = (h_src.reshape(ND, NCH, CW)
              .transpose(1, 0, 2).reshape(NCH * ND, CW))
    src_p = jnp.pad(edge_src_ds, (0, EPAD - E),
                    constant_values=ND).reshape(ROWS, 128)
    dst_p = jnp.pad(edge_dst_ds, (0, EPAD - E),
                    constant_values=0).reshape(ROWS, 128)
    zeros_n = jnp.zeros((NBOUNCE, CW), jnp.float32)
    zeros_d = jnp.zeros((DSTRIPE,), jnp.float32)

    num_flat, denom = _sc_edges(src_p, dst_p, h_flat, a_srcp,
                                a_dst.reshape(NS), zeros_n, zeros_d)
    return _epilogue(num_flat, denom, bias_ds, W_lin, b_lin)
